# baseline (device time: 22951 ns/iter reference)
import jax
import jax.numpy as jnp
from jax import lax
from jax.experimental import pallas as pl
from jax.experimental.pallas import tpu as pltpu

N_DEV = 4
N_CHUNK = 2


def kernel(x, W1, W2):
    m, k = x.shape
    h_per = W1.shape[1]
    n = W2.shape[1]
    cw = m // N_CHUNK

    def body(x_ref, w1_ref, w2_ref, out_ref,
             p_ref, recv_ref, send_sems, recv_sems):
        my_pos = lax.axis_index("i")
        pa = my_pos ^ 1
        pb = (N_DEV - 1) - my_pos
        pd = my_pos ^ 2

        barrier_sem = pltpu.get_barrier_semaphore()
        for nbr in (pa, pb, pd):
            pl.semaphore_signal(
                barrier_sem, inc=1,
                device_id=(nbr,), device_id_type=pl.DeviceIdType.MESH,
            )
        pl.semaphore_wait(barrier_sem, 3)

        xb = x_ref[...].astype(jnp.bfloat16)
        w1b = w1_ref[...].astype(jnp.bfloat16)
        w2b = w2_ref[...].astype(jnp.bfloat16)

        peers = (pa, pb, pd)

        rdmas = []
        parts = []
        for c in range(N_CHUNK):
            hc = jnp.maximum(
                jnp.dot(
                    xb[c * cw:(c + 1) * cw, :], w1b,
                    preferred_element_type=jnp.float32,
                ),
                0.0,
            ).astype(jnp.bfloat16)
            pc = jnp.dot(hc, w2b, preferred_element_type=jnp.float32)
            p_ref[c] = pc.astype(jnp.bfloat16)
            parts.append(pc)
            for j, peer in enumerate(peers):
                slot = j * N_CHUNK + c
                r = pltpu.make_async_remote_copy(
                    src_ref=p_ref.at[c],
                    dst_ref=recv_ref.at[slot],
                    send_sem=send_sems.at[slot],
                    recv_sem=recv_sems.at[slot],
                    device_id=(peer,),
                    device_id_type=pl.DeviceIdType.MESH,
                )
                r.start()
                rdmas.append(r)

        for c in range(N_CHUNK):
            acc = parts[c]
            for j in range(3):
                slot = j * N_CHUNK + c
                rdmas[c * 3 + j].wait_recv()
                acc = acc + recv_ref[slot].astype(jnp.float32)
            out_ref[c * cw:(c + 1) * cw, :] = acc

        for r in rdmas:
            r.wait_send()

    return pl.pallas_call(
        body,
        out_shape=jax.ShapeDtypeStruct((m, n), jnp.float32),
        in_specs=[
            pl.BlockSpec(memory_space=pltpu.VMEM),
            pl.BlockSpec(memory_space=pltpu.VMEM),
            pl.BlockSpec(memory_space=pltpu.VMEM),
        ],
        out_specs=pl.BlockSpec(memory_space=pltpu.VMEM),
        scratch_shapes=[
            pltpu.VMEM((N_CHUNK, m // N_CHUNK, n), jnp.bfloat16),
            pltpu.VMEM((3 * N_CHUNK, m // N_CHUNK, n), jnp.bfloat16),
            pltpu.SemaphoreType.DMA((3 * N_CHUNK,)),
            pltpu.SemaphoreType.DMA((3 * N_CHUNK,)),
        ],
        compiler_params=pltpu.CompilerParams(collective_id=0),
    )(x, W1, W2)


# device time: 18089 ns/iter; 1.2688x vs baseline; 1.2688x over previous
import jax
import jax.numpy as jnp
from jax import lax
from jax.experimental import pallas as pl
from jax.experimental.pallas import tpu as pltpu

N_DEV = 4
N_PIECE = 2


def kernel(x, W1, W2):
    m, k = x.shape
    h_per = W1.shape[1]
    n = W2.shape[1]
    gw = n // 2
    hh = m // 2
    ph = hh // N_PIECE

    def body(x_ref, w1_ref, w2_ref, out_ref,
             p_ref, half_ref, full_ref, recv_ref, send_sems, recv_sems):
        my_pos = lax.axis_index("i")
        pa = my_pos ^ 1
        pb = (N_DEV - 1) - my_pos

        barrier_sem = pltpu.get_barrier_semaphore()
        for nbr in (pa, pb):
            pl.semaphore_signal(
                barrier_sem, inc=1,
                device_id=(nbr,), device_id_type=pl.DeviceIdType.MESH,
            )
        pl.semaphore_wait(barrier_sem, 2)

        partners = [(pa, pb, pa), (pb, pa, pb)]
        keep_top = [
            (my_pos == 0) | (my_pos == 3),
            my_pos <= 1,
        ]
        k_off = [jnp.where(kt, 0, hh) for kt in keep_top]
        s_off = [hh - ko for ko in k_off]

        xb = x_ref[...].astype(jnp.bfloat16)
        w1b = w1_ref[...].astype(jnp.bfloat16)
        hb = jnp.maximum(
            jnp.dot(xb, w1b, preferred_element_type=jnp.float32), 0.0
        ).astype(jnp.bfloat16)
        w2b = w2_ref[...].astype(jnp.bfloat16)

        pieces = [(g, q) for q in range(N_PIECE) for g in range(2)]

        def slot(stage, g, q):
            return stage * 2 * N_PIECE + g * N_PIECE + q

        def copy(stage, g, q, src):
            return pltpu.make_async_remote_copy(
                src_ref=src,
                dst_ref=recv_ref.at[slot(stage, g, q)],
                send_sem=send_sems.at[slot(stage, g, q)],
                recv_sem=recv_sems.at[slot(stage, g, q)],
                device_id=(partners[g][stage],),
                device_id_type=pl.DeviceIdType.MESH,
            )

        rdma0 = {}
        for g in range(2):
            pc = jnp.dot(
                hb, w2b[:, g * gw:(g + 1) * gw],
                preferred_element_type=jnp.float32,
            )
            p_ref[g] = pc.astype(jnp.bfloat16)
            for q in range(N_PIECE):
                r = copy(0, g, q, p_ref.at[g, pl.ds(s_off[g] + q * ph, ph), :])
                r.start()
                rdma0[(g, q)] = r

        rdma1 = {}
        for g, q in pieces:
            rdma0[(g, q)].wait_recv()
            half_ref[g * N_PIECE + q] = (
                p_ref[g, pl.ds(k_off[g] + q * ph, ph), :]
                + recv_ref[slot(0, g, q)]
            )
            r = copy(1, g, q, half_ref.at[g * N_PIECE + q])
            r.start()
            rdma1[(g, q)] = r

        rdma2 = {}
        for g, q in pieces:
            rdma1[(g, q)].wait_recv()
            full = half_ref[g * N_PIECE + q] + recv_ref[slot(1, g, q)]
            full_ref[g * N_PIECE + q] = full
            r = copy(2, g, q, full_ref.at[g * N_PIECE + q])
            r.start()
            rdma2[(g, q)] = r
            out_ref[pl.ds(k_off[g] + q * ph, ph), g * gw:(g + 1) * gw] = (
                full.astype(jnp.float32)
            )

        for g, q in pieces:
            rdma2[(g, q)].wait_recv()
            out_ref[pl.ds(s_off[g] + q * ph, ph), g * gw:(g + 1) * gw] = (
                recv_ref[slot(2, g, q)].astype(jnp.float32)
            )

        for r in (*rdma0.values(), *rdma1.values(), *rdma2.values()):
            r.wait_send()

    n_slots = 3 * 2 * N_PIECE
    return pl.pallas_call(
        body,
        out_shape=jax.ShapeDtypeStruct((m, n), jnp.float32),
        in_specs=[
            pl.BlockSpec(memory_space=pltpu.VMEM),
            pl.BlockSpec(memory_space=pltpu.VMEM),
            pl.BlockSpec(memory_space=pltpu.VMEM),
        ],
        out_specs=pl.BlockSpec(memory_space=pltpu.VMEM),
        scratch_shapes=[
            pltpu.VMEM((2, m, gw), jnp.bfloat16),
            pltpu.VMEM((2 * N_PIECE, ph, gw), jnp.bfloat16),
            pltpu.VMEM((2 * N_PIECE, ph, gw), jnp.bfloat16),
            pltpu.VMEM((n_slots, ph, gw), jnp.bfloat16),
            pltpu.SemaphoreType.DMA((n_slots,)),
            pltpu.SemaphoreType.DMA((n_slots,)),
        ],
        compiler_params=pltpu.CompilerParams(collective_id=0),
    )(x, W1, W2)


# device time: 17213 ns/iter; 1.3334x vs baseline; 1.0509x over previous
import jax
import jax.numpy as jnp
from jax import lax
from jax.experimental import pallas as pl
from jax.experimental.pallas import tpu as pltpu

N_DEV = 4
N_CHUNK = 4


def kernel(x, W1, W2):
    m, k = x.shape
    h_per = W1.shape[1]
    n = W2.shape[1]

    def body(x_ref, w1_ref, w2_ref, out_ref, send_ref, recv_ref,
             send_sems, recv_sems):
        my_pos = lax.axis_index("i")
        left = (my_pos - 1) % N_DEV
        right = (my_pos + 1) % N_DEV

        barrier_sem = pltpu.get_barrier_semaphore()
        for nbr in (left, right):
            pl.semaphore_signal(
                barrier_sem, inc=1,
                device_id=(nbr,), device_id_type=pl.DeviceIdType.MESH,
            )
        pl.semaphore_wait(barrier_sem, 2)

        xb = x_ref[...].astype(jnp.bfloat16)
        w1b = w1_ref[...].astype(jnp.bfloat16)
        w2b = w2_ref[...].astype(jnp.bfloat16)

        partner_a = my_pos ^ 1
        partner_b = (N_DEV - 1) - my_pos
        cw = m // N_CHUNK

        def stage_partner(stage, c):
            if (c % 2 == 0) == (stage == 0):
                return partner_a
            return partner_b

        parts = []
        rdma_a = []
        for c in range(N_CHUNK):
            hc = jnp.maximum(
                jnp.dot(
                    xb[c * cw:(c + 1) * cw, :], w1b,
                    preferred_element_type=jnp.float32,
                ),
                0.0,
            ).astype(jnp.bfloat16)
            pc = jnp.dot(hc, w2b, preferred_element_type=jnp.float32)
            send_ref[c] = pc.astype(jnp.bfloat16)
            r = pltpu.make_async_remote_copy(
                src_ref=send_ref.at[c],
                dst_ref=recv_ref.at[c],
                send_sem=send_sems.at[c],
                recv_sem=recv_sems.at[c],
                device_id=(stage_partner(0, c),),
                device_id_type=pl.DeviceIdType.MESH,
            )
            r.start()
            parts.append(pc)
            rdma_a.append(r)

        accs = []
        rdma_b = []
        for c in range(N_CHUNK):
            rdma_a[c].wait_recv()
            acc = parts[c].astype(jnp.bfloat16) + recv_ref[c]
            send_ref[N_CHUNK + c] = acc
            r = pltpu.make_async_remote_copy(
                src_ref=send_ref.at[N_CHUNK + c],
                dst_ref=recv_ref.at[N_CHUNK + c],
                send_sem=send_sems.at[N_CHUNK + c],
                recv_sem=recv_sems.at[N_CHUNK + c],
                device_id=(stage_partner(1, c),),
                device_id_type=pl.DeviceIdType.MESH,
            )
            r.start()
            accs.append(acc)
            rdma_b.append(r)

        for c in range(N_CHUNK):
            rdma_b[c].wait_recv()
            out_ref[c * cw:(c + 1) * cw, :] = (
                accs[c] + recv_ref[N_CHUNK + c]
            ).astype(jnp.float32)

        for r in rdma_a + rdma_b:
            r.wait_send()

    return pl.pallas_call(
        body,
        out_shape=jax.ShapeDtypeStruct((m, n), jnp.float32),
        in_specs=[
            pl.BlockSpec(memory_space=pltpu.VMEM),
            pl.BlockSpec(memory_space=pltpu.VMEM),
            pl.BlockSpec(memory_space=pltpu.VMEM),
        ],
        out_specs=pl.BlockSpec(memory_space=pltpu.VMEM),
        scratch_shapes=[
            pltpu.VMEM((2 * N_CHUNK, m // N_CHUNK, n), jnp.bfloat16),
            pltpu.VMEM((2 * N_CHUNK, m // N_CHUNK, n), jnp.bfloat16),
            pltpu.SemaphoreType.DMA((2 * N_CHUNK,)),
            pltpu.SemaphoreType.DMA((2 * N_CHUNK,)),
        ],
        compiler_params=pltpu.CompilerParams(collective_id=0),
    )(x, W1, W2)
